# baseline (speedup 1.0000x reference)
import jax
import jax.numpy as jnp
from jax.experimental import pallas as pl


def _mm_block(support_ref, adj_ref, out_ref):
    out_ref[...] = jnp.dot(
        adj_ref[...], support_ref[...], preferred_element_type=jnp.float32
    )


def kernel(support, adj):
    n, d = support.shape
    bm = 80
    assert n % bm == 0
    return pl.pallas_call(
        _mm_block,
        grid=(n // bm,),
        in_specs=[
            pl.BlockSpec((n, d), lambda i: (0, 0)),
            pl.BlockSpec((bm, n), lambda i: (i, 0)),
        ],
        out_specs=pl.BlockSpec((bm, d), lambda i: (i, 0)),
        out_shape=jax.ShapeDtypeStruct((n, d), jnp.float32),
    )(support, adj)


# speedup vs baseline: 1.3411x; 1.3411x over previous
import jax
import jax.numpy as jnp
from jax.experimental import pallas as pl


def _mm_block(support_ref, adj_ref, out_ref):
    out_ref[...] = jnp.dot(
        adj_ref[...], support_ref[...], preferred_element_type=jnp.float32
    )


def kernel(support, adj):
    n, d = support.shape
    bm = 512
    grid_m = -(-n // bm)
    return pl.pallas_call(
        _mm_block,
        grid=(grid_m,),
        in_specs=[
            pl.BlockSpec((n, d), lambda i: (0, 0)),
            pl.BlockSpec((bm, n), lambda i: (i, 0)),
        ],
        out_specs=pl.BlockSpec((bm, d), lambda i: (i, 0)),
        out_shape=jax.ShapeDtypeStruct((n, d), jnp.float32),
    )(support, adj)


# speedup vs baseline: 1.3486x; 1.0056x over previous
import jax
import jax.numpy as jnp
from jax.experimental import pallas as pl


def _mm_block(support_ref, adj_ref, out_ref):
    out_ref[...] = jnp.dot(
        adj_ref[...].astype(jnp.bfloat16),
        support_ref[...].astype(jnp.bfloat16),
        preferred_element_type=jnp.float32,
    )


def kernel(support, adj):
    n, d = support.shape
    bm = 200
    grid_m = -(-n // bm)
    return pl.pallas_call(
        _mm_block,
        grid=(grid_m,),
        in_specs=[
            pl.BlockSpec((n, d), lambda i: (0, 0)),
            pl.BlockSpec((bm, n), lambda i: (i, 0)),
        ],
        out_specs=pl.BlockSpec((bm, d), lambda i: (i, 0)),
        out_shape=jax.ShapeDtypeStruct((n, d), jnp.float32),
    )(support, adj)
